# baseline (speedup 1.0000x reference)
import functools

import jax
import jax.numpy as jnp
from jax import lax
from jax.experimental import pallas as pl
from jax.experimental.pallas import tpu as pltpu
from jax.experimental.pallas import tpu_sc as plsc

B = 16384
N_FIELDS = 26
VOCAB = 100000
EMB_DIM = 32
N_CONT = 13
M_LENGTH = 128
N_EMB = N_FIELDS * EMB_DIM
N_EMB_PAD = 896

NW = 32
TOT = B * N_FIELDS
ROWS_PER_GROUP = N_EMB_PAD // EMB_DIM
N_GROUPS = B // 8
GROUPS_PER_W = N_GROUPS // NW
ROWS_PER_W = GROUPS_PER_W * 8 * ROWS_PER_GROUP

RAW_PER_W = TOT // NW
RAW_CHUNK = 128
RAW_NCHUNK = RAW_PER_W // RAW_CHUNK

G_CHUNK = 128
G_NCHUNK = ROWS_PER_W // G_CHUNK
SUP = 4
NSUP = G_NCHUNK // SUP
SUP_ROWS = SUP * G_CHUNK
SUP_TROWS = SUP_ROWS * EMB_DIM // (8 * 128)

_mesh = plsc.VectorSubcoreMesh(core_axis_name="c", subcore_axis_name="s")


@functools.partial(
    pl.kernel,
    mesh=_mesh,
    compiler_params=pltpu.CompilerParams(
        use_tc_tiling_on_sc=False, needs_layout_passes=False),
    out_type=jax.ShapeDtypeStruct((N_GROUPS * ROWS_PER_GROUP // 4, 8, 128),
                                  jnp.float32),
    scratch_types=[
        pltpu.VMEM((RAW_NCHUNK, RAW_CHUNK), jnp.int32),
        pltpu.VMEM((G_NCHUNK, G_CHUNK), jnp.int32),
        pltpu.VMEM((2, SUP_ROWS, EMB_DIM), jnp.float32),
        pltpu.VMEM((2, SUP_TROWS, 8, 128), jnp.float32),
        pltpu.SemaphoreType.DMA,
        pltpu.SemaphoreType.DMA,
    ],
)
def _sc_gather(xcat_hbm, tab_hbm, out_hbm, idx_v, idx2_v, rows_v, shaped_v,
               gsem, ssem):
    cid = lax.axis_index("c")
    sid = lax.axis_index("s")
    wid = sid * 2 + cid
    base = wid * RAW_PER_W

    pltpu.sync_copy(xcat_hbm.at[wid], idx_v)

    def _fill(j, carry):
        for k in range(G_CHUNK // 16):
            d = j * G_CHUNK + k * 16 + lax.iota(jnp.int32, 16)
            idx2_v[j, pl.ds(k * 16, 16)] = d * 181 + wid * 63
        return carry

    lax.fori_loop(0, G_NCHUNK, _fill, 0)

    inv13 = jnp.int32(-991146299)

    def _permute(j, carry):
        for k in range(RAW_CHUNK // 16):
            p = base + j * RAW_CHUNK + k * 16 + lax.iota(jnp.int32, 16)
            f = lax.rem(p, N_FIELDS)
            b = lax.shift_right_logical(p - f, 1) * inv13
            r = lax.bitwise_and(b, 7)
            g_local = lax.shift_right_logical(b, 3) - wid * GROUPS_PER_W
            dst = (g_local * ROWS_PER_GROUP * 8
                   + lax.shift_right_logical(f, 2) * 32 + r * 4
                   + lax.bitwise_and(f, 3))
            val = idx_v[j, pl.ds(k * 16, 16)] + f * VOCAB
            plsc.store_scatter(
                idx2_v,
                [lax.shift_right_logical(dst, 7), lax.bitwise_and(dst, 127)],
                val)
        return carry

    lax.fori_loop(0, RAW_NCHUNK, _permute, 0)

    def _repack(q):
        def body(iv, carry):
            t = lax.shift_right_logical(iv, 2)
            s8 = lax.bitwise_and(iv, 3) * 2
            r8 = iv * 8
            for u in range(16):
                val = rows_v[q, r8 + (u >> 1), pl.ds((u & 1) * 16, 16)]
                shaped_v[q, t, s8 + (u >> 3), pl.ds((u & 7) * 16, 16)] = val
            return carry
        lax.fori_loop(0, SUP_ROWS * EMB_DIM // 256, body, 0)

    out_base = wid * (ROWS_PER_W // 32)
    ghand = [None, None]
    pending = [None, None]

    def _drain(s):
        q = s % 2
        for g in ghand[q]:
            g.wait()
        if pending[q] is not None:
            pending[q].wait()
        _repack(q)
        pending[q] = pltpu.async_copy(
            shaped_v.at[q],
            out_hbm.at[pl.ds(out_base + s * SUP_TROWS, SUP_TROWS)],
            ssem,
        )

    for s in range(NSUP):
        p = s % 2
        ghand[p] = [
            pltpu.async_copy(
                tab_hbm.at[idx2_v.at[s * SUP + c]],
                rows_v.at[p, pl.ds(c * G_CHUNK, G_CHUNK)],
                gsem,
            )
            for c in range(SUP)
        ]
        if s >= 1:
            _drain(s - 1)
    _drain(NSUP - 1)
    for p in range(2):
        if pending[p] is not None:
            pending[p].wait()


TILE_B = 1024
GRID = B // TILE_B
TILE_G = TILE_B // 8
N_TILES_ROW = N_EMB_PAD // 128


def _mlp_body(xc_ref, emb_ref, w1e_ref, w1c_ref, b1_ref, bnw_ref, bnb_ref,
              out_ref, stat_scr):
    i = pl.program_id(0)

    @pl.when(i == 0)
    def _():
        xc = xc_ref[...]
        mean = jnp.mean(xc, axis=0)
        var = jnp.mean(xc * xc, axis=0) - mean * mean
        s = bnw_ref[...] * lax.rsqrt(var + 1e-5)
        stat_scr[0, :] = s
        stat_scr[1, :] = bnb_ref[...] - mean * s

    s = stat_scr[0, :]
    t = stat_scr[1, :]
    xcb = xc_ref[pl.ds(i * TILE_B, TILE_B), :]
    x2 = xcb * s[None, :] + t[None, :]
    acc = lax.dot_general(x2, w1c_ref[...],
                          (((1,), (1,)), ((), ())),
                          preferred_element_type=jnp.float32)
    e4 = emb_ref[...].reshape(TILE_G, N_TILES_ROW, 8, 128)
    for tt in range(N_TILES_ROW):
        et = e4[:, tt].reshape(TILE_B, 128)
        wt = w1e_ref[:, pl.ds(tt * 128, 128)]
        acc = acc + lax.dot_general(et, wt,
                                    (((1,), (1,)), ((), ())),
                                    preferred_element_type=jnp.float32)
    out_ref[...] = jnp.maximum(acc + b1_ref[...][None, :], 0.0)


_mlp = pl.pallas_call(
    _mlp_body,
    grid=(GRID,),
    in_specs=[
        pl.BlockSpec((B, N_CONT), lambda i: (0, 0)),
        pl.BlockSpec((TILE_G * N_TILES_ROW, 8, 128), lambda i: (i, 0, 0)),
        pl.BlockSpec((M_LENGTH, N_EMB_PAD), lambda i: (0, 0)),
        pl.BlockSpec((M_LENGTH, N_CONT), lambda i: (0, 0)),
        pl.BlockSpec((M_LENGTH,), lambda i: (0,)),
        pl.BlockSpec((N_CONT,), lambda i: (0,)),
        pl.BlockSpec((N_CONT,), lambda i: (0,)),
    ],
    out_specs=pl.BlockSpec((TILE_B, M_LENGTH), lambda i: (i, 0)),
    out_shape=jax.ShapeDtypeStruct((B, M_LENGTH), jnp.float32),
    scratch_shapes=[pltpu.VMEM((2, N_CONT), jnp.float32)],
)


def kernel(x_cat, x_cont, tables, W1, b1, bn_w, bn_b):
    flat_tab = tables.reshape(N_FIELDS * VOCAB, EMB_DIM)
    xcat_slabs = x_cat.astype(jnp.int32).reshape(NW, RAW_NCHUNK, RAW_CHUNK)
    emb_img = _sc_gather(xcat_slabs, flat_tab)
    w1e = W1[:, :N_EMB]
    w1e_pad = jnp.pad(w1e, ((0, 0), (0, N_EMB_PAD - N_EMB)))
    w1c = W1[:, N_EMB:]
    return _mlp(x_cont, emb_img, w1e_pad, w1c, b1, bn_w, bn_b)


# speedup vs baseline: 1.0530x; 1.0530x over previous
import functools

import jax
import jax.numpy as jnp
from jax import lax
from jax.experimental import pallas as pl
from jax.experimental.pallas import tpu as pltpu
from jax.experimental.pallas import tpu_sc as plsc

B = 16384
N_FIELDS = 26
VOCAB = 100000
EMB_DIM = 32
N_CONT = 13
M_LENGTH = 128
N_EMB = N_FIELDS * EMB_DIM
N_EMB_PAD = 896

NW = 32
TOT = B * N_FIELDS
ROWS_PER_GROUP = N_EMB_PAD // EMB_DIM
N_GROUPS = B // 8
GROUPS_PER_W = N_GROUPS // NW
ROWS_PER_W = GROUPS_PER_W * 8 * ROWS_PER_GROUP

RAW_PER_W = TOT // NW
RAW_CHUNK = 128
RAW_NCHUNK = RAW_PER_W // RAW_CHUNK

G_CHUNK = 128
G_NCHUNK = ROWS_PER_W // G_CHUNK
SUP = 8
NSUP = G_NCHUNK // SUP
SUP_ROWS = SUP * G_CHUNK
ROWS_PER_PLANE_W = GROUPS_PER_W * 32

_mesh = plsc.VectorSubcoreMesh(core_axis_name="c", subcore_axis_name="s")


@functools.partial(
    pl.kernel,
    mesh=_mesh,
    compiler_params=pltpu.CompilerParams(
        use_tc_tiling_on_sc=False, needs_layout_passes=False),
    out_type=jax.ShapeDtypeStruct((NW * ROWS_PER_W, EMB_DIM), jnp.float32),
    scratch_types=[
        pltpu.VMEM((RAW_NCHUNK, RAW_CHUNK), jnp.int32),
        pltpu.VMEM((G_NCHUNK, G_CHUNK), jnp.int32),
        pltpu.VMEM((2, SUP_ROWS, EMB_DIM), jnp.float32),
        pltpu.SemaphoreType.DMA,
        pltpu.SemaphoreType.DMA,
    ],
)
def _sc_gather(xcat_hbm, tab_hbm, out_hbm, idx_v, idx2_v, rows_v, gsem, ssem):
    cid = lax.axis_index("c")
    sid = lax.axis_index("s")
    wid = sid * 2 + cid
    base = wid * RAW_PER_W

    pltpu.sync_copy(xcat_hbm.at[wid], idx_v)

    def _fill(j, carry):
        for k in range(G_CHUNK // 16):
            d = j * G_CHUNK + k * 16 + lax.iota(jnp.int32, 16)
            idx2_v[j, pl.ds(k * 16, 16)] = d * 181 + wid * 63
        return carry

    lax.fori_loop(0, G_NCHUNK, _fill, 0)

    inv13 = jnp.int32(-991146299)

    def _permute(j, carry):
        for k in range(RAW_CHUNK // 16):
            p = base + j * RAW_CHUNK + k * 16 + lax.iota(jnp.int32, 16)
            f = lax.rem(p, N_FIELDS)
            b = lax.shift_right_logical(p - f, 1) * inv13
            r = lax.bitwise_and(b, 7)
            g_local = lax.shift_right_logical(b, 3) - wid * GROUPS_PER_W
            dst = (lax.shift_right_logical(f, 2) * ROWS_PER_PLANE_W
                   + g_local * 32 + r * 4 + lax.bitwise_and(f, 3))
            val = idx_v[j, pl.ds(k * 16, 16)] + f * VOCAB
            plsc.store_scatter(
                idx2_v,
                [lax.shift_right_logical(dst, 7), lax.bitwise_and(dst, 127)],
                val)
        return carry

    lax.fori_loop(0, RAW_NCHUNK, _permute, 0)

    ghand = [None, None]
    pending = [None, None]

    def _wb(s):
        q = s % 2
        for g in ghand[q]:
            g.wait()
        row = ((s // 2) * (NW * ROWS_PER_PLANE_W)
               + wid * ROWS_PER_PLANE_W + (s % 2) * SUP_ROWS)
        pending[q] = pltpu.async_copy(
            rows_v.at[q],
            out_hbm.at[pl.ds(row, SUP_ROWS)],
            ssem,
        )

    for s in range(NSUP):
        p = s % 2
        if pending[p] is not None:
            pending[p].wait()
            pending[p] = None
        ghand[p] = [
            pltpu.async_copy(
                tab_hbm.at[idx2_v.at[s * SUP + c]],
                rows_v.at[p, pl.ds(c * G_CHUNK, G_CHUNK)],
                gsem,
            )
            for c in range(SUP)
        ]
        if s >= 1:
            _wb(s - 1)
    _wb(NSUP - 1)
    for p in range(2):
        if pending[p] is not None:
            pending[p].wait()


TILE_B = 1024
GRID = B // TILE_B
TILE_G = TILE_B // 8
N_TILES_ROW = N_EMB_PAD // 128


def _mlp_body(xc_ref, emb_ref, w1e_ref, w1c_ref, b1_ref, bnw_ref, bnb_ref,
              out_ref, stat_scr):
    i = pl.program_id(0)

    @pl.when(i == 0)
    def _():
        xc = xc_ref[...]
        mean = jnp.mean(xc, axis=0)
        var = jnp.mean(xc * xc, axis=0) - mean * mean
        s = bnw_ref[...] * lax.rsqrt(var + 1e-5)
        stat_scr[0, :] = s
        stat_scr[1, :] = bnb_ref[...] - mean * s

    s = stat_scr[0, :]
    t = stat_scr[1, :]
    xcb = xc_ref[pl.ds(i * TILE_B, TILE_B), :]
    x2 = xcb * s[None, :] + t[None, :]
    acc = lax.dot_general(x2, w1c_ref[...],
                          (((1,), (1,)), ((), ())),
                          preferred_element_type=jnp.float32)
    e4 = emb_ref[...]
    for tt in range(N_TILES_ROW):
        et = e4[tt].reshape(TILE_B, 128)
        wt = w1e_ref[:, pl.ds(tt * 128, 128)]
        acc = acc + lax.dot_general(et, wt,
                                    (((1,), (1,)), ((), ())),
                                    preferred_element_type=jnp.float32)
    out_ref[...] = jnp.maximum(acc + b1_ref[...][None, :], 0.0)


_mlp = pl.pallas_call(
    _mlp_body,
    grid=(GRID,),
    in_specs=[
        pl.BlockSpec((B, N_CONT), lambda i: (0, 0)),
        pl.BlockSpec((N_TILES_ROW, TILE_G, 8, 128), lambda i: (0, i, 0, 0)),
        pl.BlockSpec((M_LENGTH, N_EMB_PAD), lambda i: (0, 0)),
        pl.BlockSpec((M_LENGTH, N_CONT), lambda i: (0, 0)),
        pl.BlockSpec((M_LENGTH,), lambda i: (0,)),
        pl.BlockSpec((N_CONT,), lambda i: (0,)),
        pl.BlockSpec((N_CONT,), lambda i: (0,)),
    ],
    out_specs=pl.BlockSpec((TILE_B, M_LENGTH), lambda i: (i, 0)),
    out_shape=jax.ShapeDtypeStruct((B, M_LENGTH), jnp.float32),
    scratch_shapes=[pltpu.VMEM((2, N_CONT), jnp.float32)],
)


def kernel(x_cat, x_cont, tables, W1, b1, bn_w, bn_b):
    flat_tab = tables.reshape(N_FIELDS * VOCAB, EMB_DIM)
    xcat_slabs = x_cat.astype(jnp.int32).reshape(NW, RAW_NCHUNK, RAW_CHUNK)
    emb_rows = _sc_gather(xcat_slabs, flat_tab)
    emb_img = emb_rows.reshape(N_TILES_ROW, N_GROUPS, 8, 128)
    w1e = W1[:, :N_EMB]
    w1e_pad = jnp.pad(w1e, ((0, 0), (0, N_EMB_PAD - N_EMB)))
    w1c = W1[:, N_EMB:]
    return _mlp(x_cont, emb_img, w1e_pad, w1c, b1, bn_w, bn_b)
